# baseline (speedup 1.0000x reference)
import jax
import jax.numpy as jnp
from jax import lax
from jax.experimental import pallas as pl
from jax.experimental.pallas import tpu as pltpu
from jax.experimental.pallas import tpu_sc as plsc

N = 100000
K = 16
NC, NS, L = 2, 16, 16
NW = NC * NS
TPW = 3136
CHUNK = 112
NCHUNK = TPW // CHUNK
EC = CHUNK * K
EC4 = 4 * EC
GRP = CHUNK // L


def _body(t4, idx0, idx1, outx, outy, outz,
          idxv0a, idxv1a, idx40a, idx41a, rows0a, rows1a,
          idxv0b, idxv1b, idx40b, idx41b, rows0b, rows1b,
          outv, sema, semb):
    pa = (idxv0a, idxv1a, idx40a, idx41a, rows0a, rows1a)
    pb = (idxv0b, idxv1b, idx40b, idx41b, rows0b, rows1b)
    wid = lax.axis_index("s") * NC + lax.axis_index("c")

    def base_of(c):
        return jnp.minimum(wid * TPW + c * CHUNK, N - CHUNK)

    iota4 = lax.iota(jnp.int32, L) * 4

    def build4(idxv, idx4v):
        def st(sstep, _):
            v4 = idxv[pl.ds(sstep * L, L)] * 4
            base = iota4 + sstep * (L * 4)
            plsc.store_scatter(idx4v, [base], v4)
            plsc.store_scatter(idx4v, [base + 1], v4 + 1)
            plsc.store_scatter(idx4v, [base + 2], v4 + 2)
            plsc.store_scatter(idx4v, [base + 3], v4 + 3)
            return 0

        lax.fori_loop(0, EC // L, st, 0)

    def fire(c, bufs, sem):
        idxv0, idxv1, idx40, idx41, rows0, rows1 = bufs
        edge_base = base_of(c) * K
        pltpu.sync_copy(idx0.at[pl.ds(edge_base, EC)], idxv0)
        pltpu.sync_copy(idx1.at[pl.ds(edge_base, EC)], idxv1)
        build4(idxv0, idx40)
        build4(idxv1, idx41)
        pltpu.async_copy(t4.at[idx40], rows0, sem)
        pltpu.async_copy(t4.at[idx41], rows1, sem)

    def drain(bufs, sem):
        for k in (4, 5):
            pltpu.make_async_copy(t4.at[pl.ds(0, EC4)], bufs[k], sem).wait()

    def compute(c, planes):
        def do_group(g, _):
            zero = jnp.zeros((L,), jnp.float32)
            axx = axy = axz = ayy = ayz = azz = zero
            bx = by = bz = zero
            rows0 = planes[4]
            rows1 = planes[5]
            lanes4 = lax.iota(jnp.int32, L) * (K * 4) + g * (L * K * 4)
            for j in range(K):
                e4 = lanes4 + j * 4
                x0 = plsc.load_gather(rows0, [e4])
                y0 = plsc.load_gather(rows0, [e4 + 1])
                z0 = plsc.load_gather(rows0, [e4 + 2])
                u0 = plsc.load_gather(rows0, [e4 + 3])
                x1 = plsc.load_gather(rows1, [e4])
                y1 = plsc.load_gather(rows1, [e4 + 1])
                z1 = plsc.load_gather(rows1, [e4 + 2])
                u1 = plsc.load_gather(rows1, [e4 + 3])
                dx = x0 - x1
                dy = y0 - y1
                dz = z0 - z1
                du = u0 - u1
                s = dx * dx + dy * dy + dz * dz
                w2 = jnp.where(s == 0.0, jnp.float32(1.0), jnp.float32(1.0) / s)
                wdx = w2 * dx
                wdy = w2 * dy
                wdz = w2 * dz
                axx = axx + wdx * dx
                axy = axy + wdx * dy
                axz = axz + wdx * dz
                ayy = ayy + wdy * dy
                ayz = ayz + wdy * dz
                azz = azz + wdz * dz
                bx = bx + wdx * du
                by = by + wdy * du
                bz = bz + wdz * du
            cof11 = ayy * azz - ayz * ayz
            cof12 = axy * azz - ayz * axz
            cof13 = axy * ayz - ayy * axz
            cof22 = axx * azz - axz * axz
            cof23 = axx * ayz - axy * axz
            cof33 = axx * ayy - axy * axy
            det = axx * cof11 - axy * cof12 + axz * cof13
            inv = jnp.float32(1.0) / det
            outv[pl.ds(0 * CHUNK + g * L, L)] = (bx * cof11 - by * cof12 + bz * cof13) * inv
            outv[pl.ds(1 * CHUNK + g * L, L)] = (-bx * cof12 + by * cof22 - bz * cof23) * inv
            outv[pl.ds(2 * CHUNK + g * L, L)] = (bx * cof13 - by * cof23 + bz * cof33) * inv
            return 0

        lax.fori_loop(0, GRP, do_group, 0)

        node_base = base_of(c)
        pltpu.sync_copy(outv.at[pl.ds(0 * CHUNK, CHUNK)], outx.at[pl.ds(node_base, CHUNK)])
        pltpu.sync_copy(outv.at[pl.ds(1 * CHUNK, CHUNK)], outy.at[pl.ds(node_base, CHUNK)])
        pltpu.sync_copy(outv.at[pl.ds(2 * CHUNK, CHUNK)], outz.at[pl.ds(node_base, CHUNK)])

    fire(0, pa, sema)

    def do_pair(i, _):
        ca = 2 * i
        fire(ca + 1, pb, semb)
        drain(pa, sema)
        compute(ca, pa)

        @pl.when(ca + 2 < NCHUNK)
        def _():
            fire(ca + 2, pa, sema)

        drain(pb, semb)
        compute(ca + 1, pb)
        return 0

    lax.fori_loop(0, NCHUNK // 2, do_pair, 0)


@jax.jit
def _run(t4, idx0, idx1):
    mesh = plsc.VectorSubcoreMesh(
        core_axis_name="c", subcore_axis_name="s", num_cores=NC, num_subcores=NS
    )
    o = jax.ShapeDtypeStruct((N,), jnp.float32)
    return pl.kernel(
        _body,
        out_type=[o, o, o],
        mesh=mesh,
        compiler_params=pltpu.CompilerParams(needs_layout_passes=False),
        scratch_types=(
            [pltpu.VMEM((EC,), jnp.int32)] * 2
            + [pltpu.VMEM((EC4,), jnp.int32)] * 2
            + [pltpu.VMEM((EC4,), jnp.float32)] * 2
            + [pltpu.VMEM((EC,), jnp.int32)] * 2
            + [pltpu.VMEM((EC4,), jnp.int32)] * 2
            + [pltpu.VMEM((EC4,), jnp.float32)] * 2
            + [pltpu.VMEM((3 * CHUNK,), jnp.float32)]
            + [pltpu.SemaphoreType.DMA] * 2
        ),
    )(t4, idx0, idx1)


def kernel(coordinates, u, connectivity_tensor):
    t4 = jnp.concatenate([coordinates, u], axis=1).reshape(4 * N)
    conn = connectivity_tensor.astype(jnp.int32)
    i0 = conn[:, :, 0].reshape(N * K)
    i1 = conn[:, :, 1].reshape(N * K)
    outx, outy, outz = _run(t4, i0, i1)
    return (outx[:, None], outy[:, None], outz[:, None])


# speedup vs baseline: 1.0785x; 1.0785x over previous
import jax
import jax.numpy as jnp
from jax import lax
from jax.experimental import pallas as pl
from jax.experimental.pallas import tpu as pltpu
from jax.experimental.pallas import tpu_sc as plsc

N = 100000
K = 16
NC, NS, L = 2, 16, 16
NW = NC * NS
TPW = 3136
CHUNK = 224
NCHUNK = TPW // CHUNK
EC = CHUNK * K
GRP = CHUNK // L


def _body(xs, ys, zs, us, idx0, idx1, outx, outy, outz,
          idxv0a, idxv1a, x0a, y0a, z0a, u0a, x1a, y1a, z1a, u1a,
          idxv0b, idxv1b, x0b, y0b, z0b, u0b, x1b, y1b, z1b, u1b,
          outv, sema, semb):
    pa = (x0a, y0a, z0a, u0a, x1a, y1a, z1a, u1a)
    pb = (x0b, y0b, z0b, u0b, x1b, y1b, z1b, u1b)
    wid = lax.axis_index("s") * NC + lax.axis_index("c")

    def base_of(c):
        return jnp.minimum(wid * TPW + c * CHUNK, N - CHUNK)

    def fire(c, idxv0, idxv1, planes, sem):
        edge_base = base_of(c) * K
        pltpu.sync_copy(idx0.at[pl.ds(edge_base, EC)], idxv0)
        pltpu.sync_copy(idx1.at[pl.ds(edge_base, EC)], idxv1)
        H = EC // 2
        for half in range(2):
            h0 = half * H
            pltpu.async_copy(xs.at[idxv0.at[pl.ds(h0, H)]], planes[0].at[pl.ds(h0, H)], sem)
            pltpu.async_copy(ys.at[idxv0.at[pl.ds(h0, H)]], planes[1].at[pl.ds(h0, H)], sem)
            pltpu.async_copy(zs.at[idxv0.at[pl.ds(h0, H)]], planes[2].at[pl.ds(h0, H)], sem)
            pltpu.async_copy(us.at[idxv0.at[pl.ds(h0, H)]], planes[3].at[pl.ds(h0, H)], sem)
            pltpu.async_copy(xs.at[idxv1.at[pl.ds(h0, H)]], planes[4].at[pl.ds(h0, H)], sem)
            pltpu.async_copy(ys.at[idxv1.at[pl.ds(h0, H)]], planes[5].at[pl.ds(h0, H)], sem)
            pltpu.async_copy(zs.at[idxv1.at[pl.ds(h0, H)]], planes[6].at[pl.ds(h0, H)], sem)
            pltpu.async_copy(us.at[idxv1.at[pl.ds(h0, H)]], planes[7].at[pl.ds(h0, H)], sem)

    def drain(planes, sem):
        for k in range(8):
            pltpu.make_async_copy(xs.at[pl.ds(0, EC)], planes[k], sem).wait()

    def compute(c, planes):
        def do_group(g, _):
            zero = jnp.zeros((L,), jnp.float32)
            axx = axy = axz = ayy = ayz = azz = zero
            bx = by = bz = zero
            lanes = lax.iota(jnp.int32, L) * K + g * (L * K)
            for j in range(K):
                e = lanes + j
                x0 = plsc.load_gather(planes[0], [e])
                y0 = plsc.load_gather(planes[1], [e])
                z0 = plsc.load_gather(planes[2], [e])
                u0 = plsc.load_gather(planes[3], [e])
                x1 = plsc.load_gather(planes[4], [e])
                y1 = plsc.load_gather(planes[5], [e])
                z1 = plsc.load_gather(planes[6], [e])
                u1 = plsc.load_gather(planes[7], [e])
                dx = x0 - x1
                dy = y0 - y1
                dz = z0 - z1
                du = u0 - u1
                s = dx * dx + dy * dy + dz * dz
                w2 = jnp.where(s == 0.0, jnp.float32(1.0), jnp.float32(1.0) / s)
                wdx = w2 * dx
                wdy = w2 * dy
                wdz = w2 * dz
                axx = axx + wdx * dx
                axy = axy + wdx * dy
                axz = axz + wdx * dz
                ayy = ayy + wdy * dy
                ayz = ayz + wdy * dz
                azz = azz + wdz * dz
                bx = bx + wdx * du
                by = by + wdy * du
                bz = bz + wdz * du
            cof11 = ayy * azz - ayz * ayz
            cof12 = axy * azz - ayz * axz
            cof13 = axy * ayz - ayy * axz
            cof22 = axx * azz - axz * axz
            cof23 = axx * ayz - axy * axz
            cof33 = axx * ayy - axy * axy
            det = axx * cof11 - axy * cof12 + axz * cof13
            inv = jnp.float32(1.0) / det
            outv[pl.ds(0 * CHUNK + g * L, L)] = (bx * cof11 - by * cof12 + bz * cof13) * inv
            outv[pl.ds(1 * CHUNK + g * L, L)] = (-bx * cof12 + by * cof22 - bz * cof23) * inv
            outv[pl.ds(2 * CHUNK + g * L, L)] = (bx * cof13 - by * cof23 + bz * cof33) * inv
            return 0

        lax.fori_loop(0, GRP, do_group, 0)

        node_base = base_of(c)
        pltpu.sync_copy(outv.at[pl.ds(0 * CHUNK, CHUNK)], outx.at[pl.ds(node_base, CHUNK)])
        pltpu.sync_copy(outv.at[pl.ds(1 * CHUNK, CHUNK)], outy.at[pl.ds(node_base, CHUNK)])
        pltpu.sync_copy(outv.at[pl.ds(2 * CHUNK, CHUNK)], outz.at[pl.ds(node_base, CHUNK)])

    fire(0, idxv0a, idxv1a, pa, sema)

    def do_pair(i, _):
        ca = 2 * i
        fire(ca + 1, idxv0b, idxv1b, pb, semb)
        drain(pa, sema)
        compute(ca, pa)

        @pl.when(ca + 2 < NCHUNK)
        def _():
            fire(ca + 2, idxv0a, idxv1a, pa, sema)

        drain(pb, semb)
        compute(ca + 1, pb)
        return 0

    lax.fori_loop(0, NCHUNK // 2, do_pair, 0)


@jax.jit
def _run(xs, ys, zs, us, idx0, idx1):
    mesh = plsc.VectorSubcoreMesh(
        core_axis_name="c", subcore_axis_name="s", num_cores=NC, num_subcores=NS
    )
    o = jax.ShapeDtypeStruct((N,), jnp.float32)
    return pl.kernel(
        _body,
        out_type=[o, o, o],
        mesh=mesh,
        compiler_params=pltpu.CompilerParams(needs_layout_passes=False),
        scratch_types=(
            [pltpu.VMEM((EC,), jnp.int32)] * 2
            + [pltpu.VMEM((EC,), jnp.float32)] * 8
            + [pltpu.VMEM((EC,), jnp.int32)] * 2
            + [pltpu.VMEM((EC,), jnp.float32)] * 8
            + [pltpu.VMEM((3 * CHUNK,), jnp.float32)]
            + [pltpu.SemaphoreType.DMA] * 2
        ),
    )(xs, ys, zs, us, idx0, idx1)


def kernel(coordinates, u, connectivity_tensor):
    xs = coordinates[:, 0]
    ys = coordinates[:, 1]
    zs = coordinates[:, 2]
    us = u[:, 0]
    conn = connectivity_tensor.astype(jnp.int32)
    i0 = conn[:, :, 0].reshape(N * K)
    i1 = conn[:, :, 1].reshape(N * K)
    outx, outy, outz = _run(xs, ys, zs, us, i0, i1)
    return (outx[:, None], outy[:, None], outz[:, None])
